# Initial kernel scaffold; baseline (speedup 1.0000x reference)
import jax
import jax.numpy as jnp
from jax.experimental import pallas as pl


def kernel(inputs, kernel):
    raise NotImplementedError("write your pallas kernel here")


# speedup vs baseline: 3.9059x; 3.9059x over previous
import jax
import jax.numpy as jnp
from jax import lax
from jax.experimental import pallas as pl

_NUM_PILOT = 512


def _mask_mul_body(x_ref, w_ref, o_ref):
    w = w_ref[...]
    u = lax.bitcast_convert_type(w, jnp.uint32)
    top = jnp.uint32(0x80000000)
    key = jnp.where(u >= top, ~u, u | top)

    def body(i, p):
        sh = (jnp.uint32(31) - i.astype(jnp.uint32))
        cand = p | lax.shift_left(jnp.uint32(1), sh)
        cnt = jnp.sum((key >= cand).astype(jnp.int32))
        return jnp.where(cnt >= _NUM_PILOT, cand, p)

    p = lax.fori_loop(0, 32, body, jnp.uint32(0))
    t = jnp.where(p >= top, p ^ top, ~p)
    thresh = lax.bitcast_convert_type(t, jnp.float32)
    mask = (w > thresh).astype(jnp.float32)
    o_ref[...] = x_ref[...] * mask


def kernel(inputs, kernel):
    out = pl.pallas_call(
        _mask_mul_body,
        out_shape=jax.ShapeDtypeStruct(inputs.shape, inputs.dtype),
    )(inputs, kernel)
    return out
